# Initial kernel scaffold; baseline (speedup 1.0000x reference)
import jax
import jax.numpy as jnp
from jax.experimental import pallas as pl


def kernel(x, edge_index, W1, b1, W2, b2, W3, b3):
    raise NotImplementedError("write your pallas kernel here")


# speedup vs baseline: 30.0357x; 30.0357x over previous
import functools

import jax
import jax.numpy as jnp
from jax import lax
from jax.experimental import pallas as pl
from jax.experimental.pallas import tpu as pltpu
from jax.experimental.pallas import tpu_sc as plsc

N = 10000
E = 320000
NC = 2
NS = 16
NW = NC * NS
CHUNK = 128
KC = 79
EPAD = NW * KC * CHUNK
NPAD = 10240
RPT = NPAD // NS


def _make_agg(H):
    mesh = plsc.VectorSubcoreMesh(core_axis_name="c", subcore_axis_name="s")

    @functools.partial(
        pl.kernel,
        out_type=jax.ShapeDtypeStruct((NC, NPAD, H), jnp.float32),
        mesh=mesh,
        scratch_types=[
            pltpu.VMEM((KC, CHUNK), jnp.int32),
            pltpu.VMEM((KC, CHUNK), jnp.int32),
            pltpu.VMEM((2, CHUNK, H), jnp.float32),
            pltpu.VMEM_SHARED((NPAD, H), jnp.float32),
            pltpu.SemaphoreType.DMA,
            pltpu.SemaphoreType.DMA,
        ],
        compiler_params=pltpu.CompilerParams(use_tc_tiling_on_sc=False),
    )
    def agg(hs_hbm, row_hbm, col_hbm, zeros_hbm, out_hbm,
            row_v, col_v, msg_v, acc, gsem, ssem):
        c = lax.axis_index("c")
        s = lax.axis_index("s")
        wid = c * NS + s
        pltpu.sync_copy(row_hbm.at[wid], row_v)
        pltpu.sync_copy(col_hbm.at[wid], col_v)
        pltpu.sync_copy(zeros_hbm, acc.at[pl.ds(s * RPT, RPT)])
        plsc.subcore_barrier()

        def body(j, carry):
            pltpu.async_copy(hs_hbm.at[row_v.at[j]], msg_v.at[0], gsem).wait()
            pltpu.async_copy(msg_v.at[0], acc.at[col_v.at[j]], ssem,
                             add=True).wait()
            return carry

        lax.fori_loop(0, KC, body, 0)
        plsc.subcore_barrier()
        pltpu.sync_copy(acc.at[pl.ds(s * RPT, RPT)],
                        out_hbm.at[c, pl.ds(s * RPT, RPT)])

    return agg


DEGW = 16


def _make_deg():
    mesh = plsc.VectorSubcoreMesh(core_axis_name="c", subcore_axis_name="s")

    @functools.partial(
        pl.kernel,
        out_type=jax.ShapeDtypeStruct((NC, NPAD, DEGW), jnp.float32),
        mesh=mesh,
        scratch_types=[
            pltpu.VMEM((KC, CHUNK), jnp.int32),
            pltpu.VMEM((CHUNK, DEGW), jnp.float32),
            pltpu.VMEM_SHARED((NPAD, DEGW), jnp.float32),
            pltpu.SemaphoreType.DMA,
        ],
        compiler_params=pltpu.CompilerParams(use_tc_tiling_on_sc=False),
    )
    def deg(ones_hbm, col_hbm, zeros_hbm, out_hbm, col_v, ones_v, acc, sem):
        c = lax.axis_index("c")
        s = lax.axis_index("s")
        wid = c * NS + s
        pltpu.sync_copy(col_hbm.at[wid], col_v)
        pltpu.sync_copy(ones_hbm, ones_v)
        pltpu.sync_copy(zeros_hbm, acc.at[pl.ds(s * RPT, RPT)])
        plsc.subcore_barrier()

        def body(j, carry):
            pltpu.async_copy(ones_v, acc.at[col_v.at[j]], sem, add=True).wait()
            return carry

        lax.fori_loop(0, KC, body, 0)
        plsc.subcore_barrier()
        pltpu.sync_copy(acc.at[pl.ds(s * RPT, RPT)],
                        out_hbm.at[c, pl.ds(s * RPT, RPT)])

    return deg


_agg32 = _make_agg(32)
_agg16 = _make_agg(16)
_deg = _make_deg()


def _pre_body(x_ref, w_ref, degp_ref, hs_ref, dis_ref):
    deg = degp_ref[0, :N, 0:1] + degp_ref[1, :N, 0:1] + 1.0
    dis = lax.rsqrt(deg)
    h = jnp.dot(x_ref[...], w_ref[...], preferred_element_type=jnp.float32)
    hs_ref[...] = dis * h
    dis_ref[...] = dis


def _mid_body(p_ref, hs_ref, dis_ref, b_ref, w_ref, hsn_ref):
    agg = p_ref[0, :N] + p_ref[1, :N] + hs_ref[...]
    out = dis_ref[...] * agg + b_ref[...]
    hsn_ref[...] = dis_ref[...] * jnp.dot(
        out, w_ref[...], preferred_element_type=jnp.float32)


def _fin_body(p_ref, hs_ref, dis_ref, b_ref, o_ref):
    z = dis_ref[...] * (p_ref[0, :N] + p_ref[1, :N] + hs_ref[...]) + b_ref[...]
    m = jnp.max(z, axis=1, keepdims=True)
    e = jnp.exp(z - m)
    o_ref[...] = z - (jnp.log(jnp.sum(e, axis=1, keepdims=True)) + m)


def _pre_call(x, w, degp):
    return pl.pallas_call(
        _pre_body,
        out_shape=(jax.ShapeDtypeStruct((N, w.shape[1]), jnp.float32),
                   jax.ShapeDtypeStruct((N, 1), jnp.float32)),
    )(x, w, degp)


def _mid_call(p, hs, dis, b, w):
    return pl.pallas_call(
        _mid_body,
        out_shape=jax.ShapeDtypeStruct((N, w.shape[1]), jnp.float32),
    )(p, hs, dis, b, w)


def _fin_call(p, hs, dis, b):
    return pl.pallas_call(
        _fin_body,
        out_shape=jax.ShapeDtypeStruct((N, hs.shape[1]), jnp.float32),
    )(p, hs, dis, b)


def kernel(x, edge_index, W1, b1, W2, b2, W3, b3):
    row = edge_index[0]
    col = edge_index[1]
    npad = EPAD - E
    pr = (jnp.arange(npad, dtype=jnp.int32) * 97) % jnp.int32(N)
    pc = jnp.int32(N) + (jnp.arange(npad, dtype=jnp.int32) % jnp.int32(NPAD - N))
    row3 = jnp.concatenate([row, pr]).reshape(NW, KC, CHUNK)
    col3 = jnp.concatenate([col, pc]).reshape(NW, KC, CHUNK)

    ones_c = jnp.ones((CHUNK, DEGW), jnp.float32)
    z1 = jnp.zeros((RPT, DEGW), jnp.float32)
    z32 = jnp.zeros((RPT, 32), jnp.float32)
    z16 = jnp.zeros((RPT, 16), jnp.float32)

    degp = _deg(ones_c, col3, z1)
    hs1, dis = _pre_call(x, W1, degp)
    p1 = _agg32(hs1, row3, col3, z32)
    hs2 = _mid_call(p1, hs1, dis, b1.reshape(1, -1), W2)
    p2 = _agg16(hs2, row3, col3, z16)
    hs3 = _mid_call(p2, hs2, dis, b2.reshape(1, -1), W3)
    p3 = _agg16(hs3, row3, col3, z16)
    return _fin_call(p3, hs3, dis, b3.reshape(1, -1))


# speedup vs baseline: 53.0199x; 1.7652x over previous
import functools

import jax
import jax.numpy as jnp
from jax import lax
from jax.experimental import pallas as pl
from jax.experimental.pallas import tpu as pltpu
from jax.experimental.pallas import tpu_sc as plsc

N = 10000
E = 320000
NC = 2
NS = 16
NW = NC * NS
CHUNK = 128
NBUF = 8
KO = 10
KC = NBUF * KO
EPAD = NW * KC * CHUNK
NPAD = 10240
RPT = NPAD // NS


def _make_agg(H):
    mesh = plsc.VectorSubcoreMesh(core_axis_name="c", subcore_axis_name="s")

    @functools.partial(
        pl.kernel,
        out_type=jax.ShapeDtypeStruct((NC, NPAD, H), jnp.float32),
        mesh=mesh,
        scratch_types=[
            pltpu.VMEM((KC, CHUNK), jnp.int32),
            pltpu.VMEM((KC, CHUNK), jnp.int32),
            pltpu.VMEM((2 * NBUF, CHUNK, H), jnp.float32),
            pltpu.VMEM_SHARED((NPAD, H), jnp.float32),
            pltpu.SemaphoreType.DMA,
            pltpu.SemaphoreType.DMA,
        ],
        compiler_params=pltpu.CompilerParams(use_tc_tiling_on_sc=False),
    )
    def agg(hs_hbm, row_hbm, col_hbm, zeros_hbm, out_hbm,
            row_v, col_v, msg_v, acc, gsem, ssem):
        c = lax.axis_index("c")
        s = lax.axis_index("s")
        wid = c * NS + s
        pltpu.sync_copy(row_hbm.at[wid], row_v)
        pltpu.sync_copy(col_hbm.at[wid], col_v)
        pltpu.sync_copy(zeros_hbm, acc.at[pl.ds(s * RPT, RPT)])
        plsc.subcore_barrier()

        for b in range(NBUF):
            pltpu.async_copy(hs_hbm.at[row_v.at[b]], msg_v.at[b], gsem)

        def body(g, carry):
            sel = (g % 2) * NBUF
            nxt = ((g + 1) % 2) * NBUF

            @pl.when(g + 1 < KO)
            def _():
                for b in range(NBUF):
                    jj = (g + 1) * NBUF + b
                    pltpu.async_copy(hs_hbm.at[row_v.at[jj]],
                                     msg_v.at[nxt + b], gsem)

            for b in range(NBUF):
                j = g * NBUF + b
                pltpu.make_async_copy(hs_hbm.at[row_v.at[j]],
                                      msg_v.at[sel + b], gsem).wait()
                pltpu.async_copy(msg_v.at[sel + b], acc.at[col_v.at[j]],
                                 ssem, add=True)
            for b in range(NBUF):
                j = g * NBUF + b
                pltpu.make_async_copy(msg_v.at[sel + b], acc.at[col_v.at[j]],
                                      ssem).wait()
            return carry

        lax.fori_loop(0, KO, body, 0)
        plsc.subcore_barrier()
        pltpu.sync_copy(acc.at[pl.ds(s * RPT, RPT)],
                        out_hbm.at[c, pl.ds(s * RPT, RPT)])

    return agg


DEGW = 16


def _make_deg():
    mesh = plsc.VectorSubcoreMesh(core_axis_name="c", subcore_axis_name="s")

    @functools.partial(
        pl.kernel,
        out_type=jax.ShapeDtypeStruct((NC, NPAD, DEGW), jnp.float32),
        mesh=mesh,
        scratch_types=[
            pltpu.VMEM((KC, CHUNK), jnp.int32),
            pltpu.VMEM((CHUNK, DEGW), jnp.float32),
            pltpu.VMEM_SHARED((NPAD, DEGW), jnp.float32),
            pltpu.SemaphoreType.DMA,
        ],
        compiler_params=pltpu.CompilerParams(use_tc_tiling_on_sc=False),
    )
    def deg(ones_hbm, col_hbm, zeros_hbm, out_hbm, col_v, ones_v, acc, sem):
        c = lax.axis_index("c")
        s = lax.axis_index("s")
        wid = c * NS + s
        pltpu.sync_copy(col_hbm.at[wid], col_v)
        pltpu.sync_copy(ones_hbm, ones_v)
        pltpu.sync_copy(zeros_hbm, acc.at[pl.ds(s * RPT, RPT)])
        plsc.subcore_barrier()

        def body(g, carry):
            for b in range(NBUF):
                j = g * NBUF + b
                pltpu.async_copy(ones_v, acc.at[col_v.at[j]], sem, add=True)
            for b in range(NBUF):
                j = g * NBUF + b
                pltpu.make_async_copy(ones_v, acc.at[col_v.at[j]], sem).wait()
            return carry

        lax.fori_loop(0, KO, body, 0)
        plsc.subcore_barrier()
        pltpu.sync_copy(acc.at[pl.ds(s * RPT, RPT)],
                        out_hbm.at[c, pl.ds(s * RPT, RPT)])

    return deg


_agg32 = _make_agg(32)
_agg16 = _make_agg(16)
_deg = _make_deg()


def _pre_body(x_ref, w_ref, degp_ref, hs_ref, dis_ref):
    deg = degp_ref[0, :N, 0:1] + degp_ref[1, :N, 0:1] + 1.0
    dis = lax.rsqrt(deg)
    h = jnp.dot(x_ref[...], w_ref[...], preferred_element_type=jnp.float32)
    hs_ref[...] = dis * h
    dis_ref[...] = dis


def _mid_body(p_ref, hs_ref, dis_ref, b_ref, w_ref, hsn_ref):
    agg = p_ref[0, :N] + p_ref[1, :N] + hs_ref[...]
    out = dis_ref[...] * agg + b_ref[...]
    hsn_ref[...] = dis_ref[...] * jnp.dot(
        out, w_ref[...], preferred_element_type=jnp.float32)


def _fin_body(p_ref, hs_ref, dis_ref, b_ref, o_ref):
    z = dis_ref[...] * (p_ref[0, :N] + p_ref[1, :N] + hs_ref[...]) + b_ref[...]
    m = jnp.max(z, axis=1, keepdims=True)
    e = jnp.exp(z - m)
    o_ref[...] = z - (jnp.log(jnp.sum(e, axis=1, keepdims=True)) + m)


def _pre_call(x, w, degp):
    return pl.pallas_call(
        _pre_body,
        out_shape=(jax.ShapeDtypeStruct((N, w.shape[1]), jnp.float32),
                   jax.ShapeDtypeStruct((N, 1), jnp.float32)),
    )(x, w, degp)


def _mid_call(p, hs, dis, b, w):
    return pl.pallas_call(
        _mid_body,
        out_shape=jax.ShapeDtypeStruct((N, w.shape[1]), jnp.float32),
    )(p, hs, dis, b, w)


def _fin_call(p, hs, dis, b):
    return pl.pallas_call(
        _fin_body,
        out_shape=jax.ShapeDtypeStruct((N, hs.shape[1]), jnp.float32),
    )(p, hs, dis, b)


def kernel(x, edge_index, W1, b1, W2, b2, W3, b3):
    row = edge_index[0]
    col = edge_index[1]
    npad = EPAD - E
    pr = (jnp.arange(npad, dtype=jnp.int32) * 97) % jnp.int32(N)
    pc = jnp.int32(N) + (jnp.arange(npad, dtype=jnp.int32) % jnp.int32(NPAD - N))
    row3 = jnp.concatenate([row, pr]).reshape(NW, KC, CHUNK)
    col3 = jnp.concatenate([col, pc]).reshape(NW, KC, CHUNK)

    ones_c = jnp.ones((CHUNK, DEGW), jnp.float32)
    z1 = jnp.zeros((RPT, DEGW), jnp.float32)
    z32 = jnp.zeros((RPT, 32), jnp.float32)
    z16 = jnp.zeros((RPT, 16), jnp.float32)

    degp = _deg(ones_c, col3, z1)
    hs1, dis = _pre_call(x, W1, degp)
    p1 = _agg32(hs1, row3, col3, z32)
    hs2 = _mid_call(p1, hs1, dis, b1.reshape(1, -1), W2)
    p2 = _agg16(hs2, row3, col3, z16)
    hs3 = _mid_call(p2, hs2, dis, b2.reshape(1, -1), W3)
    p3 = _agg16(hs3, row3, col3, z16)
    return _fin_call(p3, hs3, dis, b3.reshape(1, -1))


# speedup vs baseline: 66.2807x; 1.2501x over previous
import functools

import jax
import jax.numpy as jnp
from jax import lax
from jax.experimental import pallas as pl
from jax.experimental.pallas import tpu as pltpu
from jax.experimental.pallas import tpu_sc as plsc

N = 10000
E = 320000
NC = 2
NS = 16
NW = NC * NS
CHUNK = 128
NBUF = 6
KO = 13
KC = NBUF * KO
NCHUNK = E // CHUNK
LEFT0 = NW * KC
NLEFT = NCHUNK - LEFT0
RPT = N // NS


def _make_agg(H):
    mesh = plsc.VectorSubcoreMesh(core_axis_name="c", subcore_axis_name="s")

    @functools.partial(
        pl.kernel,
        out_type=jax.ShapeDtypeStruct((NC, N, H), jnp.float32),
        mesh=mesh,
        scratch_types=[
            pltpu.VMEM((KC, CHUNK), jnp.int32),
            pltpu.VMEM((KC, CHUNK), jnp.int32),
            pltpu.VMEM((1, CHUNK), jnp.int32),
            pltpu.VMEM((1, CHUNK), jnp.int32),
            pltpu.VMEM((2 * NBUF, CHUNK, H), jnp.float32),
            pltpu.VMEM_SHARED((N, H), jnp.float32),
            pltpu.SemaphoreType.DMA,
            pltpu.SemaphoreType.DMA,
        ],
        compiler_params=pltpu.CompilerParams(use_tc_tiling_on_sc=False),
    )
    def agg(hs_hbm, eidx_hbm, zeros_hbm, out_hbm,
            row_v, col_v, lrow_v, lcol_v, msg_v, acc, gsem, ssem):
        c = lax.axis_index("c")
        s = lax.axis_index("s")
        wid = c * NS + s
        pltpu.sync_copy(eidx_hbm.at[0, pl.ds(wid * KC, KC)], row_v)
        pltpu.sync_copy(eidx_hbm.at[1, pl.ds(wid * KC, KC)], col_v)

        @pl.when(wid < NLEFT)
        def _():
            pltpu.sync_copy(eidx_hbm.at[0, pl.ds(LEFT0 + wid, 1)], lrow_v)
            pltpu.sync_copy(eidx_hbm.at[1, pl.ds(LEFT0 + wid, 1)], lcol_v)

        pltpu.sync_copy(zeros_hbm, acc.at[pl.ds(s * RPT, RPT)])
        plsc.subcore_barrier()

        for b in range(NBUF):
            pltpu.async_copy(hs_hbm.at[row_v.at[b]], msg_v.at[b], gsem)

        def body(g, carry):
            sel = (g % 2) * NBUF
            nxt = ((g + 1) % 2) * NBUF

            @pl.when(g + 1 < KO)
            def _():
                for b in range(NBUF):
                    jj = (g + 1) * NBUF + b
                    pltpu.async_copy(hs_hbm.at[row_v.at[jj]],
                                     msg_v.at[nxt + b], gsem)

            for b in range(NBUF):
                j = g * NBUF + b
                pltpu.make_async_copy(hs_hbm.at[row_v.at[j]],
                                      msg_v.at[sel + b], gsem).wait()
                pltpu.async_copy(msg_v.at[sel + b], acc.at[col_v.at[j]],
                                 ssem, add=True)
            for b in range(NBUF):
                j = g * NBUF + b
                pltpu.make_async_copy(msg_v.at[sel + b], acc.at[col_v.at[j]],
                                      ssem).wait()
            return carry

        lax.fori_loop(0, KO, body, 0)

        @pl.when(wid < NLEFT)
        def _():
            pltpu.async_copy(hs_hbm.at[lrow_v.at[0]], msg_v.at[0], gsem).wait()
            pltpu.async_copy(msg_v.at[0], acc.at[lcol_v.at[0]], ssem,
                             add=True).wait()

        plsc.subcore_barrier()
        pltpu.sync_copy(acc.at[pl.ds(s * RPT, RPT)],
                        out_hbm.at[c, pl.ds(s * RPT, RPT)])

    return agg


def _make_deg():
    mesh = plsc.VectorSubcoreMesh(core_axis_name="c", subcore_axis_name="s")

    @functools.partial(
        pl.kernel,
        out_type=(jax.ShapeDtypeStruct((NC, N, 16), jnp.float32),
                  jax.ShapeDtypeStruct((NC, N, 32), jnp.float32)),
        mesh=mesh,
        scratch_types=[
            pltpu.VMEM((KC, CHUNK), jnp.int32),
            pltpu.VMEM((1, CHUNK), jnp.int32),
            pltpu.VMEM((CHUNK, 16), jnp.float32),
            pltpu.VMEM((CHUNK, 32), jnp.float32),
            pltpu.VMEM_SHARED((N, 16), jnp.float32),
            pltpu.VMEM_SHARED((N, 32), jnp.float32),
            pltpu.SemaphoreType.DMA,
            pltpu.SemaphoreType.DMA,
        ],
        compiler_params=pltpu.CompilerParams(use_tc_tiling_on_sc=False),
    )
    def deg(ones16_hbm, ones32_hbm, eidx_hbm, z16_hbm, z32_hbm,
            out16_hbm, out32_hbm,
            col_v, lcol_v, ones16_v, ones32_v, acc16, acc32, sem16, sem32):
        c = lax.axis_index("c")
        s = lax.axis_index("s")
        wid = c * NS + s
        pltpu.sync_copy(eidx_hbm.at[1, pl.ds(wid * KC, KC)], col_v)

        @pl.when(wid < NLEFT)
        def _():
            pltpu.sync_copy(eidx_hbm.at[1, pl.ds(LEFT0 + wid, 1)], lcol_v)

        pltpu.sync_copy(ones16_hbm, ones16_v)
        pltpu.sync_copy(ones32_hbm, ones32_v)
        pltpu.sync_copy(z16_hbm, acc16.at[pl.ds(s * RPT, RPT)])
        pltpu.sync_copy(z32_hbm, acc32.at[pl.ds(s * RPT, RPT)])
        plsc.subcore_barrier()

        def body(g, carry):
            for b in range(NBUF):
                j = g * NBUF + b
                pltpu.async_copy(ones16_v, acc16.at[col_v.at[j]], sem16,
                                 add=True)
                pltpu.async_copy(ones32_v, acc32.at[col_v.at[j]], sem32,
                                 add=True)
            for b in range(NBUF):
                j = g * NBUF + b
                pltpu.make_async_copy(ones16_v, acc16.at[col_v.at[j]],
                                      sem16).wait()
                pltpu.make_async_copy(ones32_v, acc32.at[col_v.at[j]],
                                      sem32).wait()
            return carry

        lax.fori_loop(0, KO, body, 0)

        @pl.when(wid < NLEFT)
        def _():
            pltpu.async_copy(ones16_v, acc16.at[lcol_v.at[0]], sem16,
                             add=True).wait()
            pltpu.async_copy(ones32_v, acc32.at[lcol_v.at[0]], sem32,
                             add=True).wait()

        plsc.subcore_barrier()
        pltpu.sync_copy(acc16.at[pl.ds(s * RPT, RPT)],
                        out16_hbm.at[c, pl.ds(s * RPT, RPT)])
        pltpu.sync_copy(acc32.at[pl.ds(s * RPT, RPT)],
                        out32_hbm.at[c, pl.ds(s * RPT, RPT)])

    return deg


_agg32 = _make_agg(32)
_agg16 = _make_agg(16)
_deg = _make_deg()


def _pre_body(x_ref, w_ref, dp16_ref, dp32_ref, hs_ref, d32_ref, d16_ref):
    d16 = lax.rsqrt(dp16_ref[0] + dp16_ref[1] + 1.0)
    d32 = lax.rsqrt(dp32_ref[0] + dp32_ref[1] + 1.0)
    h = jnp.dot(x_ref[0::4, :], w_ref[0], preferred_element_type=jnp.float32)
    for a in range(1, 4):
        h = h + jnp.dot(x_ref[a::4, :], w_ref[a],
                        preferred_element_type=jnp.float32)
    hs_ref[...] = d32 * h
    d32_ref[...] = d32
    d16_ref[...] = d16


def _mid1_body(p_ref, hs_ref, d32_ref, d16_ref, b_ref, wlo_ref, whi_ref,
               hsn_ref, out_ref):
    out_ref[...] = (d32_ref[...] * (p_ref[0] + p_ref[1] + hs_ref[...])
                    + b_ref[...])
    m = (jnp.dot(out_ref[0::2, :], wlo_ref[...],
                 preferred_element_type=jnp.float32)
         + jnp.dot(out_ref[1::2, :], whi_ref[...],
                   preferred_element_type=jnp.float32))
    hsn_ref[...] = d16_ref[...] * m


def _mid2_body(p_ref, hs_ref, d16_ref, b_ref, wbd_ref, hsn_ref):
    out = d16_ref[...] * (p_ref[0] + p_ref[1] + hs_ref[...]) + b_ref[...]
    m = jnp.dot(out, wbd_ref[...], preferred_element_type=jnp.float32)
    hsn_ref[...] = d16_ref[...] * m


def _fin_body(p_ref, hs_ref, d16_ref, b_ref, g_ref, o_ref):
    z = d16_ref[...] * (p_ref[0] + p_ref[1] + hs_ref[...]) + b_ref[...]
    m = jnp.max(z, axis=1, keepdims=True)
    e = jnp.exp(z - m)
    s = jnp.dot(e, g_ref[...], preferred_element_type=jnp.float32)
    o_ref[...] = (z - m) - jnp.log(s)


def kernel(x, edge_index, W1, b1, W2, b2, W3, b3):
    eidx = edge_index.reshape(2, NCHUNK, CHUNK)

    ones16 = jnp.ones((CHUNK, 16), jnp.float32)
    ones32 = jnp.ones((CHUNK, 32), jnp.float32)
    z32 = jnp.zeros((RPT, 32), jnp.float32)
    z16 = jnp.zeros((RPT, 16), jnp.float32)

    b1p = jnp.tile(b1, 4)[None, :]
    b2p = jnp.tile(b2, 8)[None, :]
    b3p = jnp.tile(b3, 8)[None, :]
    w1p = jnp.zeros((4, 128, 128), jnp.float32)
    for a in range(4):
        w1p = w1p.at[a, :, 32 * a:32 * (a + 1)].set(W1)
    w2bd = jax.scipy.linalg.block_diag(W2, W2, W2, W2)
    zpad = jnp.zeros((128, 64), jnp.float32)
    w2lo = jnp.concatenate([w2bd, zpad], axis=1)
    w2hi = jnp.concatenate([zpad, w2bd], axis=1)
    w3bd = jax.scipy.linalg.block_diag(*([W3] * 8))
    g16 = jnp.kron(jnp.eye(8, dtype=jnp.float32),
                   jnp.ones((16, 16), jnp.float32))

    dp16, dp32 = _deg(ones16, ones32, eidx, z16, z32)
    hs1p, d32, d16 = pl.pallas_call(
        _pre_body,
        out_shape=(jax.ShapeDtypeStruct((N // 4, 128), jnp.float32),
                   jax.ShapeDtypeStruct((N // 4, 128), jnp.float32),
                   jax.ShapeDtypeStruct((N // 8, 128), jnp.float32)),
    )(x, w1p, dp16.reshape(NC, N * 16 // 128, 128),
      dp32.reshape(NC, N * 32 // 128, 128))

    p1 = _agg32(hs1p.reshape(N, 32), eidx, z32)
    hs2p = pl.pallas_call(
        _mid1_body,
        out_shape=jax.ShapeDtypeStruct((N // 8, 128), jnp.float32),
        scratch_shapes=[pltpu.VMEM((N // 4, 128), jnp.float32)],
    )(p1.reshape(NC, N * 32 // 128, 128), hs1p, d32, d16, b1p, w2lo, w2hi)

    p2 = _agg16(hs2p.reshape(N, 16), eidx, z16)
    hs3p = pl.pallas_call(
        _mid2_body,
        out_shape=jax.ShapeDtypeStruct((N // 8, 128), jnp.float32),
    )(p2.reshape(NC, N * 16 // 128, 128), hs2p, d16, b2p, w3bd)

    p3 = _agg16(hs3p.reshape(N, 16), eidx, z16)
    outp = pl.pallas_call(
        _fin_body,
        out_shape=jax.ShapeDtypeStruct((N // 8, 128), jnp.float32),
    )(p3.reshape(NC, N * 16 // 128, 128), hs3p, d16, b3p, g16)
    return outp.reshape(N, 16)


# speedup vs baseline: 76.2872x; 1.1510x over previous
import functools

import jax
import jax.numpy as jnp
from jax import lax
from jax.experimental import pallas as pl
from jax.experimental.pallas import tpu as pltpu
from jax.experimental.pallas import tpu_sc as plsc

N = 10000
E = 320000
NC = 2
NS = 16
NW = NC * NS
CHUNK = 128
NBUF = 6
KO = 13
KC = NBUF * KO
NCHUNK = E // CHUNK
LEFT0 = NW * KC
NLEFT = NCHUNK - LEFT0
RPT = N // NS


def _make_agg(H):
    mesh = plsc.VectorSubcoreMesh(core_axis_name="c", subcore_axis_name="s")

    @functools.partial(
        pl.kernel,
        out_type=jax.ShapeDtypeStruct((NC, N, H), jnp.float32),
        mesh=mesh,
        scratch_types=[
            pltpu.VMEM((KC, CHUNK), jnp.int32),
            pltpu.VMEM((KC, CHUNK), jnp.int32),
            pltpu.VMEM((1, CHUNK), jnp.int32),
            pltpu.VMEM((1, CHUNK), jnp.int32),
            pltpu.VMEM((2 * NBUF, CHUNK, H), jnp.float32),
            pltpu.VMEM_SHARED((N, H), jnp.float32),
            pltpu.SemaphoreType.DMA,
            pltpu.SemaphoreType.DMA,
        ],
        compiler_params=pltpu.CompilerParams(use_tc_tiling_on_sc=False),
    )
    def agg(hs_hbm, eidx_hbm, zeros_hbm, out_hbm,
            row_v, col_v, lrow_v, lcol_v, msg_v, acc, gsem, ssem):
        c = lax.axis_index("c")
        s = lax.axis_index("s")
        wid = c * NS + s
        pltpu.sync_copy(eidx_hbm.at[0, pl.ds(wid * KC, KC)], row_v)
        pltpu.sync_copy(eidx_hbm.at[1, pl.ds(wid * KC, KC)], col_v)

        @pl.when(wid < NLEFT)
        def _():
            pltpu.sync_copy(eidx_hbm.at[0, pl.ds(LEFT0 + wid, 1)], lrow_v)
            pltpu.sync_copy(eidx_hbm.at[1, pl.ds(LEFT0 + wid, 1)], lcol_v)

        pltpu.sync_copy(zeros_hbm, acc.at[pl.ds(s * RPT, RPT)])
        plsc.subcore_barrier()

        for b in range(NBUF):
            pltpu.async_copy(hs_hbm.at[row_v.at[b]], msg_v.at[b], gsem)

        def body(g, carry):
            sel = (g % 2) * NBUF
            nxt = ((g + 1) % 2) * NBUF

            @pl.when(g + 1 < KO)
            def _():
                for b in range(NBUF):
                    jj = (g + 1) * NBUF + b
                    pltpu.async_copy(hs_hbm.at[row_v.at[jj]],
                                     msg_v.at[nxt + b], gsem)

            for b in range(NBUF):
                j = g * NBUF + b
                pltpu.make_async_copy(hs_hbm.at[row_v.at[j]],
                                      msg_v.at[sel + b], gsem).wait()
                pltpu.async_copy(msg_v.at[sel + b], acc.at[col_v.at[j]],
                                 ssem, add=True)
            for b in range(NBUF):
                j = g * NBUF + b
                pltpu.make_async_copy(msg_v.at[sel + b], acc.at[col_v.at[j]],
                                      ssem).wait()
            return carry

        lax.fori_loop(0, KO, body, 0)

        @pl.when(wid < NLEFT)
        def _():
            pltpu.async_copy(hs_hbm.at[lrow_v.at[0]], msg_v.at[0], gsem).wait()
            pltpu.async_copy(msg_v.at[0], acc.at[lcol_v.at[0]], ssem,
                             add=True).wait()

        plsc.subcore_barrier()
        pltpu.sync_copy(acc.at[pl.ds(s * RPT, RPT)],
                        out_hbm.at[c, pl.ds(s * RPT, RPT)])

    return agg


def _make_deg():
    mesh = plsc.VectorSubcoreMesh(core_axis_name="c", subcore_axis_name="s")

    @functools.partial(
        pl.kernel,
        out_type=jax.ShapeDtypeStruct((NC, N, 16), jnp.float32),
        mesh=mesh,
        scratch_types=[
            pltpu.VMEM((KC, CHUNK), jnp.int32),
            pltpu.VMEM((1, CHUNK), jnp.int32),
            pltpu.VMEM((CHUNK, 16), jnp.float32),
            pltpu.VMEM_SHARED((N, 16), jnp.float32),
            pltpu.SemaphoreType.DMA,
        ],
        compiler_params=pltpu.CompilerParams(use_tc_tiling_on_sc=False),
    )
    def deg(ones_hbm, eidx_hbm, z16_hbm, out_hbm,
            col_v, lcol_v, ones_v, acc, sem):
        c = lax.axis_index("c")
        s = lax.axis_index("s")
        wid = c * NS + s
        pltpu.sync_copy(eidx_hbm.at[1, pl.ds(wid * KC, KC)], col_v)

        @pl.when(wid < NLEFT)
        def _():
            pltpu.sync_copy(eidx_hbm.at[1, pl.ds(LEFT0 + wid, 1)], lcol_v)

        pltpu.sync_copy(ones_hbm, ones_v)
        pltpu.sync_copy(z16_hbm, acc.at[pl.ds(s * RPT, RPT)])
        plsc.subcore_barrier()

        def body(g, carry):
            for b in range(NBUF):
                j = g * NBUF + b
                pltpu.async_copy(ones_v, acc.at[col_v.at[j]], sem, add=True)
            for b in range(NBUF):
                j = g * NBUF + b
                pltpu.make_async_copy(ones_v, acc.at[col_v.at[j]], sem).wait()
            return carry

        lax.fori_loop(0, KO, body, 0)

        @pl.when(wid < NLEFT)
        def _():
            pltpu.async_copy(ones_v, acc.at[lcol_v.at[0]], sem, add=True).wait()

        plsc.subcore_barrier()
        pltpu.sync_copy(acc.at[pl.ds(s * RPT, RPT)],
                        out_hbm.at[c, pl.ds(s * RPT, RPT)])

    return deg


_agg32 = _make_agg(32)
_agg16 = _make_agg(16)
_deg = _make_deg()


def _mm1_body(x_ref, w_ref, h_ref):
    h = jnp.dot(x_ref[0::4, :], w_ref[0], preferred_element_type=jnp.float32)
    for a in range(1, 4):
        h = h + jnp.dot(x_ref[a::4, :], w_ref[a],
                        preferred_element_type=jnp.float32)
    h_ref[...] = h


def _scale_body(h_ref, dp_ref, k8_ref, e4_ref, hs_ref, d32_ref, d16_ref):
    d16 = lax.rsqrt(dp_ref[0] + dp_ref[1] + 1.0)
    d16_ref[...] = d16
    v = jnp.dot(d16, k8_ref[...], preferred_element_type=jnp.float32)
    d32_ref[0::2, :] = jnp.dot(v[:, 0:4], e4_ref[...],
                               preferred_element_type=jnp.float32)
    d32_ref[1::2, :] = jnp.dot(v[:, 4:8], e4_ref[...],
                               preferred_element_type=jnp.float32)
    hs_ref[...] = d32_ref[...] * h_ref[...]


def _mid1_body(p_ref, hs_ref, d32_ref, d16_ref, b_ref, wlo_ref, whi_ref,
               hsn_ref, out_ref):
    out_ref[...] = (d32_ref[...] * (p_ref[0] + p_ref[1] + hs_ref[...])
                    + b_ref[...])
    m = (jnp.dot(out_ref[0::2, :], wlo_ref[...],
                 preferred_element_type=jnp.float32)
         + jnp.dot(out_ref[1::2, :], whi_ref[...],
                   preferred_element_type=jnp.float32))
    hsn_ref[...] = d16_ref[...] * m


def _mid2_body(p_ref, hs_ref, d16_ref, b_ref, wbd_ref, hsn_ref):
    out = d16_ref[...] * (p_ref[0] + p_ref[1] + hs_ref[...]) + b_ref[...]
    m = jnp.dot(out, wbd_ref[...], preferred_element_type=jnp.float32)
    hsn_ref[...] = d16_ref[...] * m


def _fin_body(p_ref, hs_ref, d16_ref, b_ref, g_ref, o_ref):
    z = d16_ref[...] * (p_ref[0] + p_ref[1] + hs_ref[...]) + b_ref[...]
    m = jnp.max(z, axis=1, keepdims=True)
    e = jnp.exp(z - m)
    s = jnp.dot(e, g_ref[...], preferred_element_type=jnp.float32)
    r = (z - m) - jnp.log(s)
    for a in range(8):
        o_ref[a::8, :] = r[:, 16 * a:16 * (a + 1)]


def kernel(x, edge_index, W1, b1, W2, b2, W3, b3):
    eidx = edge_index.reshape(2, NCHUNK, CHUNK)

    ones16 = jnp.ones((CHUNK, 16), jnp.float32)
    z32 = jnp.zeros((RPT, 32), jnp.float32)
    z16 = jnp.zeros((RPT, 16), jnp.float32)

    b1p = jnp.tile(b1, 4)[None, :]
    b2p = jnp.tile(b2, 8)[None, :]
    b3p = jnp.tile(b3, 8)[None, :]
    w1p = jnp.zeros((4, 128, 128), jnp.float32)
    for a in range(4):
        w1p = w1p.at[a, :, 32 * a:32 * (a + 1)].set(W1)
    w2bd = jax.scipy.linalg.block_diag(W2, W2, W2, W2)
    zpad = jnp.zeros((128, 64), jnp.float32)
    w2lo = jnp.concatenate([w2bd, zpad], axis=1)
    w2hi = jnp.concatenate([zpad, w2bd], axis=1)
    w3bd = jax.scipy.linalg.block_diag(*([W3] * 8))
    g16 = jnp.kron(jnp.eye(8, dtype=jnp.float32),
                   jnp.ones((16, 16), jnp.float32))
    lane = jnp.arange(128)
    k8 = ((lane[:, None] // 16 == jnp.arange(8)[None, :])
          .astype(jnp.float32) / 16.0)
    e4 = (lane[None, :] // 32 == jnp.arange(4)[:, None]).astype(jnp.float32)

    dp = _deg(ones16, eidx, z16)
    hp = pl.pallas_call(
        _mm1_body,
        out_shape=jax.ShapeDtypeStruct((N // 4, 128), jnp.float32),
    )(x, w1p)
    hs1p, d32, d16 = pl.pallas_call(
        _scale_body,
        out_shape=(jax.ShapeDtypeStruct((N // 4, 128), jnp.float32),
                   jax.ShapeDtypeStruct((N // 4, 128), jnp.float32),
                   jax.ShapeDtypeStruct((N // 8, 128), jnp.float32)),
    )(hp, dp.reshape(NC, N * 16 // 128, 128), k8, e4)

    p1 = _agg32(hs1p.reshape(N, 32), eidx, z32)
    hs2p = pl.pallas_call(
        _mid1_body,
        out_shape=jax.ShapeDtypeStruct((N // 8, 128), jnp.float32),
        scratch_shapes=[pltpu.VMEM((N // 4, 128), jnp.float32)],
    )(p1.reshape(NC, N * 32 // 128, 128), hs1p, d32, d16, b1p, w2lo, w2hi)

    p2 = _agg16(hs2p.reshape(N, 16), eidx, z16)
    hs3p = pl.pallas_call(
        _mid2_body,
        out_shape=jax.ShapeDtypeStruct((N // 8, 128), jnp.float32),
    )(p2.reshape(NC, N * 16 // 128, 128), hs2p, d16, b2p, w3bd)

    p3 = _agg16(hs3p.reshape(N, 16), eidx, z16)
    return pl.pallas_call(
        _fin_body,
        out_shape=jax.ShapeDtypeStruct((N, 16), jnp.float32),
    )(p3.reshape(NC, N * 16 // 128, 128), hs3p, d16, b3p, g16)


# speedup vs baseline: 76.4812x; 1.0025x over previous
import functools

import jax
import jax.numpy as jnp
from jax import lax
from jax.experimental import pallas as pl
from jax.experimental.pallas import tpu as pltpu
from jax.experimental.pallas import tpu_sc as plsc

N = 10000
E = 320000
NC = 2
NS = 16
NW = NC * NS
CHUNK = 128
RING = 13
GLA = 7
SLAG = 6
KC = 78
NCHUNK = E // CHUNK
LEFT0 = NW * KC
NLEFT = NCHUNK - LEFT0
RPT = N // NS


def _make_agg(H):
    mesh = plsc.VectorSubcoreMesh(core_axis_name="c", subcore_axis_name="s")

    @functools.partial(
        pl.kernel,
        out_type=jax.ShapeDtypeStruct((NC, N, H), jnp.float32),
        mesh=mesh,
        scratch_types=[
            pltpu.VMEM((KC, CHUNK), jnp.int32),
            pltpu.VMEM((KC, CHUNK), jnp.int32),
            pltpu.VMEM((1, CHUNK), jnp.int32),
            pltpu.VMEM((1, CHUNK), jnp.int32),
            pltpu.VMEM((RING, CHUNK, H), jnp.float32),
            pltpu.VMEM_SHARED((N, H), jnp.float32),
            pltpu.SemaphoreType.DMA,
            pltpu.SemaphoreType.DMA,
        ],
        compiler_params=pltpu.CompilerParams(use_tc_tiling_on_sc=False),
    )
    def agg(hs_hbm, eidx_hbm, zeros_hbm, out_hbm,
            row_v, col_v, lrow_v, lcol_v, msg_v, acc, gsem, ssem):
        c = lax.axis_index("c")
        s = lax.axis_index("s")
        wid = c * NS + s
        pltpu.sync_copy(eidx_hbm.at[0, pl.ds(wid * KC, KC)], row_v)
        pltpu.sync_copy(eidx_hbm.at[1, pl.ds(wid * KC, KC)], col_v)

        @pl.when(wid < NLEFT)
        def _():
            pltpu.sync_copy(eidx_hbm.at[0, pl.ds(LEFT0 + wid, 1)], lrow_v)
            pltpu.sync_copy(eidx_hbm.at[1, pl.ds(LEFT0 + wid, 1)], lcol_v)

        pltpu.sync_copy(zeros_hbm, acc.at[pl.ds(s * RPT, RPT)])
        plsc.subcore_barrier()

        for b in range(GLA):
            pltpu.async_copy(hs_hbm.at[row_v.at[b]], msg_v.at[b], gsem)

        def body(j, carry):
            bj = j % RING
            pltpu.make_async_copy(hs_hbm.at[row_v.at[j]],
                                  msg_v.at[bj], gsem).wait()
            pltpu.async_copy(msg_v.at[bj], acc.at[col_v.at[j]], ssem,
                             add=True)

            @pl.when(j >= SLAG)
            def _():
                jd = j - SLAG
                pltpu.make_async_copy(msg_v.at[jd % RING],
                                      acc.at[col_v.at[jd]], ssem).wait()

            @pl.when(j + GLA < KC)
            def _():
                jg = j + GLA
                pltpu.async_copy(hs_hbm.at[row_v.at[jg]],
                                 msg_v.at[jg % RING], gsem)
            return carry

        lax.fori_loop(0, KC, body, 0)
        for t in range(KC - SLAG, KC):
            pltpu.make_async_copy(msg_v.at[t % RING], acc.at[col_v.at[t]],
                                  ssem).wait()

        @pl.when(wid < NLEFT)
        def _():
            pltpu.async_copy(hs_hbm.at[lrow_v.at[0]], msg_v.at[0], gsem).wait()
            pltpu.async_copy(msg_v.at[0], acc.at[lcol_v.at[0]], ssem,
                             add=True).wait()

        plsc.subcore_barrier()
        pltpu.sync_copy(acc.at[pl.ds(s * RPT, RPT)],
                        out_hbm.at[c, pl.ds(s * RPT, RPT)])

    return agg


def _make_deg():
    mesh = plsc.VectorSubcoreMesh(core_axis_name="c", subcore_axis_name="s")

    @functools.partial(
        pl.kernel,
        out_type=jax.ShapeDtypeStruct((NC, N, 16), jnp.float32),
        mesh=mesh,
        scratch_types=[
            pltpu.VMEM((KC, CHUNK), jnp.int32),
            pltpu.VMEM((1, CHUNK), jnp.int32),
            pltpu.VMEM((CHUNK, 16), jnp.float32),
            pltpu.VMEM_SHARED((N, 16), jnp.float32),
            pltpu.SemaphoreType.DMA,
        ],
        compiler_params=pltpu.CompilerParams(use_tc_tiling_on_sc=False),
    )
    def deg(ones_hbm, eidx_hbm, z16_hbm, out_hbm,
            col_v, lcol_v, ones_v, acc, sem):
        c = lax.axis_index("c")
        s = lax.axis_index("s")
        wid = c * NS + s
        pltpu.sync_copy(eidx_hbm.at[1, pl.ds(wid * KC, KC)], col_v)

        @pl.when(wid < NLEFT)
        def _():
            pltpu.sync_copy(eidx_hbm.at[1, pl.ds(LEFT0 + wid, 1)], lcol_v)

        pltpu.sync_copy(ones_hbm, ones_v)
        pltpu.sync_copy(z16_hbm, acc.at[pl.ds(s * RPT, RPT)])
        plsc.subcore_barrier()

        def body(j, carry):
            pltpu.async_copy(ones_v, acc.at[col_v.at[j]], sem, add=True)

            @pl.when(j >= 12)
            def _():
                pltpu.make_async_copy(ones_v, acc.at[col_v.at[j - 12]],
                                      sem).wait()
            return carry

        lax.fori_loop(0, KC, body, 0)
        for t in range(KC - 12, KC):
            pltpu.make_async_copy(ones_v, acc.at[col_v.at[t]], sem).wait()

        @pl.when(wid < NLEFT)
        def _():
            pltpu.async_copy(ones_v, acc.at[lcol_v.at[0]], sem, add=True).wait()

        plsc.subcore_barrier()
        pltpu.sync_copy(acc.at[pl.ds(s * RPT, RPT)],
                        out_hbm.at[c, pl.ds(s * RPT, RPT)])

    return deg


_agg32 = _make_agg(32)
_agg16 = _make_agg(16)
_deg = _make_deg()


def _mm1_body(x_ref, w_ref, h_ref):
    h = jnp.dot(x_ref[0::4, :], w_ref[0], preferred_element_type=jnp.float32)
    for a in range(1, 4):
        h = h + jnp.dot(x_ref[a::4, :], w_ref[a],
                        preferred_element_type=jnp.float32)
    h_ref[...] = h


def _scale_body(h_ref, dp_ref, k8_ref, e4_ref, hs_ref, d32_ref, d16_ref):
    d16 = lax.rsqrt(dp_ref[0] + dp_ref[1] + 1.0)
    d16_ref[...] = d16
    v = jnp.dot(d16, k8_ref[...], preferred_element_type=jnp.float32)
    d32_ref[0::2, :] = jnp.dot(v[:, 0:4], e4_ref[...],
                               preferred_element_type=jnp.float32)
    d32_ref[1::2, :] = jnp.dot(v[:, 4:8], e4_ref[...],
                               preferred_element_type=jnp.float32)
    hs_ref[...] = d32_ref[...] * h_ref[...]


def _mid1_body(p_ref, hs_ref, d32_ref, d16_ref, b_ref, wlo_ref, whi_ref,
               hsn_ref, out_ref):
    out_ref[...] = (d32_ref[...] * (p_ref[0] + p_ref[1] + hs_ref[...])
                    + b_ref[...])
    m = (jnp.dot(out_ref[0::2, :], wlo_ref[...],
                 preferred_element_type=jnp.float32)
         + jnp.dot(out_ref[1::2, :], whi_ref[...],
                   preferred_element_type=jnp.float32))
    hsn_ref[...] = d16_ref[...] * m


def _mid2_body(p_ref, hs_ref, d16_ref, b_ref, wbd_ref, hsn_ref):
    out = d16_ref[...] * (p_ref[0] + p_ref[1] + hs_ref[...]) + b_ref[...]
    m = jnp.dot(out, wbd_ref[...], preferred_element_type=jnp.float32)
    hsn_ref[...] = d16_ref[...] * m


def _fin_body(p_ref, hs_ref, d16_ref, b_ref, g_ref, o_ref):
    z = d16_ref[...] * (p_ref[0] + p_ref[1] + hs_ref[...]) + b_ref[...]
    m = jnp.max(z, axis=1, keepdims=True)
    e = jnp.exp(z - m)
    s = jnp.dot(e, g_ref[...], preferred_element_type=jnp.float32)
    r = (z - m) - jnp.log(s)
    for a in range(8):
        o_ref[a::8, :] = r[:, 16 * a:16 * (a + 1)]


def kernel(x, edge_index, W1, b1, W2, b2, W3, b3):
    eidx = edge_index.reshape(2, NCHUNK, CHUNK)

    ones16 = jnp.ones((CHUNK, 16), jnp.float32)
    z32 = jnp.zeros((RPT, 32), jnp.float32)
    z16 = jnp.zeros((RPT, 16), jnp.float32)

    b1p = jnp.tile(b1, 4)[None, :]
    b2p = jnp.tile(b2, 8)[None, :]
    b3p = jnp.tile(b3, 8)[None, :]
    w1p = jnp.zeros((4, 128, 128), jnp.float32)
    for a in range(4):
        w1p = w1p.at[a, :, 32 * a:32 * (a + 1)].set(W1)
    w2bd = jax.scipy.linalg.block_diag(W2, W2, W2, W2)
    zpad = jnp.zeros((128, 64), jnp.float32)
    w2lo = jnp.concatenate([w2bd, zpad], axis=1)
    w2hi = jnp.concatenate([zpad, w2bd], axis=1)
    w3bd = jax.scipy.linalg.block_diag(*([W3] * 8))
    g16 = jnp.kron(jnp.eye(8, dtype=jnp.float32),
                   jnp.ones((16, 16), jnp.float32))
    lane = jnp.arange(128)
    k8 = ((lane[:, None] // 16 == jnp.arange(8)[None, :])
          .astype(jnp.float32) / 16.0)
    e4 = (lane[None, :] // 32 == jnp.arange(4)[:, None]).astype(jnp.float32)

    dp = _deg(ones16, eidx, z16)
    hp = pl.pallas_call(
        _mm1_body,
        out_shape=jax.ShapeDtypeStruct((N // 4, 128), jnp.float32),
    )(x, w1p)
    hs1p, d32, d16 = pl.pallas_call(
        _scale_body,
        out_shape=(jax.ShapeDtypeStruct((N // 4, 128), jnp.float32),
                   jax.ShapeDtypeStruct((N // 4, 128), jnp.float32),
                   jax.ShapeDtypeStruct((N // 8, 128), jnp.float32)),
    )(hp, dp.reshape(NC, N * 16 // 128, 128), k8, e4)

    p1 = _agg32(hs1p.reshape(N, 32), eidx, z32)
    hs2p = pl.pallas_call(
        _mid1_body,
        out_shape=jax.ShapeDtypeStruct((N // 8, 128), jnp.float32),
        scratch_shapes=[pltpu.VMEM((N // 4, 128), jnp.float32)],
    )(p1.reshape(NC, N * 32 // 128, 128), hs1p, d32, d16, b1p, w2lo, w2hi)

    p2 = _agg16(hs2p.reshape(N, 16), eidx, z16)
    hs3p = pl.pallas_call(
        _mid2_body,
        out_shape=jax.ShapeDtypeStruct((N // 8, 128), jnp.float32),
    )(p2.reshape(NC, N * 16 // 128, 128), hs2p, d16, b2p, w3bd)

    p3 = _agg16(hs3p.reshape(N, 16), eidx, z16)
    return pl.pallas_call(
        _fin_body,
        out_shape=jax.ShapeDtypeStruct((N, 16), jnp.float32),
    )(p3.reshape(NC, N * 16 // 128, 128), hs3p, d16, b3p, g16)
